# Initial kernel scaffold; baseline (speedup 1.0000x reference)
import jax
import jax.numpy as jnp
from jax.experimental import pallas as pl


def kernel(user_table, item_table, alpha_W, alpha_b, Wq, bq, Wk, bk, Wv, bv, Wo, bo, ln_g, ln_b, edge_index, users, pos_items, neg_items):
    raise NotImplementedError("write your pallas kernel here")


# speedup vs baseline: 4.3341x; 4.3341x over previous
import functools

import jax
import jax.numpy as jnp
import numpy as np
from jax import lax
from jax.experimental import pallas as pl
from jax.experimental.pallas import tpu as pltpu
from jax.experimental.pallas import tpu_sc as plsc

NU = 5000
NI = 5000
D = 128
NB = 3
NL = 2
E = 320000
H = 4
DH = D // H
BATCH = 4096
SU = 5120

NC = 2
NS = 16
NW = NC * NS
EPT = E // NW
CH = 80
NCHUNK = EPT // CH
TOTCH = NB * NCHUNK

_mesh = plsc.VectorSubcoreMesh(core_axis_name="c", subcore_axis_name="s",
                               num_cores=NC, num_subcores=NS)


def _edge_pass1(ut_ref, it_ref, ui_ref, ii_ref, aw_ref, ab_ref,
                e_out, awd_out, sp_out,
                uiv, iiv, gidx, giidx, sidx, ue, ie, ebuf, awbuf, aWbv, abv,
                zb, s_sh):
  cid = lax.axis_index("c")
  sid = lax.axis_index("s")
  wid = cid * NS + sid

  pltpu.sync_copy(aw_ref, aWbv)
  pltpu.sync_copy(ab_ref, abv)

  def zbody(i, _):
    zb[pl.ds(i * 16, 16)] = jnp.zeros((16,), jnp.float32)
    return 0
  lax.fori_loop(0, (NB * SU) // (NS * 16), zbody, 0)
  pltpu.sync_copy(zb, s_sh.at[pl.ds(sid * ((NB * SU) // NS), (NB * SU) // NS)])
  plsc.subcore_barrier()

  def chunk_body(c, _):
    b = c // NCHUNK
    cc = c - b * NCHUNK
    eoff = pl.multiple_of(b * E + wid * EPT + cc * CH, 8)
    pltpu.sync_copy(ui_ref.at[pl.ds(eoff, CH)], uiv)
    pltpu.sync_copy(ii_ref.at[pl.ds(eoff, CH)], iiv)
    for g in range(CH // 16):
      sl = pl.ds(g * 16, 16)
      u16 = uiv[sl]
      i16 = iiv[sl]
      gidx[sl] = u16 + b * NU
      giidx[sl] = i16 + b * NI
      sidx[sl] = u16 + b * SU
    pltpu.sync_copy(ut_ref.at[gidx], ue)
    pltpu.sync_copy(it_ref.at[giidx], ie)
    ab16 = abv[b, :]
    for g in range(CH // 16):
      eidx = jnp.arange(16, dtype=jnp.int32) + (g * 16)

      def dbody(i, carry):
        a0, a1, w0, w1 = carry
        for k in range(4):
          d = i * 4 + k
          dv = jnp.zeros((16,), jnp.int32) + d
          uu = plsc.load_gather(ue, [eidx, dv])
          vv = plsc.load_gather(ie, [eidx, dv])
          df = jnp.abs(uu - vv)
          wv = aWbv[b * D + d, :]
          if k % 2 == 0:
            a0 = a0 + df * wv
            w0 = w0 + df * df
          else:
            a1 = a1 + df * wv
            w1 = w1 + df * df
        return (a0, a1, w0, w1)

      z = jnp.zeros((16,), jnp.float32)
      a0, a1, w0, w1 = lax.fori_loop(0, D // 4, dbody, (z, z, z, z))
      act = a0 + a1 + ab16
      el = jnp.where(act > 0.0, act, jnp.exp(act) - 1.0)
      sl = pl.ds(g * 16, 16)
      ebuf[sl] = jnp.exp(el)
      awbuf[sl] = w0 + w1
    pltpu.sync_copy(ebuf, e_out.at[pl.ds(eoff, CH)])
    pltpu.sync_copy(awbuf, awd_out.at[pl.ds(eoff, CH)])
    pltpu.sync_copy(ebuf, s_sh.at[sidx], add=True)
    return 0

  lax.fori_loop(0, TOTCH, chunk_body, 0)
  plsc.subcore_barrier()

  @pl.when(sid == 0)
  def _():
    pltpu.sync_copy(s_sh, sp_out.at[pl.ds(cid * (NB * SU), NB * SU)])


def _edge_pass2(ut_ref, ui_ref, ii_ref, e_ref, sp_ref,
                tw_out, Sp_out,
                uiv, iiv, gidx, iidx, rows, ebuf, twbuf, rinv, stmp, zb, S_sh):
  cid = lax.axis_index("c")
  sid = lax.axis_index("s")
  wid = cid * NS + sid

  pltpu.sync_copy(sp_ref.at[pl.ds(0, NB * SU)], rinv)
  pltpu.sync_copy(sp_ref.at[pl.ds(NB * SU, NB * SU)], stmp)

  def rbody(i, _):
    sl = pl.ds(i * 16, 16)
    rinv[sl] = 1.0 / (rinv[sl] + stmp[sl])
    return 0
  lax.fori_loop(0, (NB * SU) // 16, rbody, 0)

  def zbody(i, _):
    r = i // 8
    k = i - r * 8
    zb[r, pl.ds(k * 16, 16)] = jnp.zeros((16,), jnp.float32)
    return 0
  lax.fori_loop(0, 80 * 8, zbody, 0)

  rows_per_tile = SU // NS

  for b in range(NB):
    def z2body(j, _):
      pltpu.sync_copy(zb, S_sh.at[pl.ds(sid * rows_per_tile + j * 80, 80), :])
      return 0
    lax.fori_loop(0, rows_per_tile // 80, z2body, 0)
    plsc.subcore_barrier()

    def chunk_body(cc, _):
      eoff = pl.multiple_of(b * E + wid * EPT + cc * CH, 8)
      pltpu.sync_copy(ui_ref.at[pl.ds(eoff, CH)], uiv)
      pltpu.sync_copy(ii_ref.at[pl.ds(eoff, CH)], iiv)
      pltpu.sync_copy(e_ref.at[pl.ds(eoff, CH)], ebuf)
      for g in range(CH // 16):
        sl = pl.ds(g * 16, 16)
        gidx[sl] = uiv[sl] + b * NU
        iidx[sl] = iiv[sl]
      pltpu.sync_copy(ut_ref.at[gidx], rows)
      for g in range(CH // 16):
        sl = pl.ds(g * 16, 16)
        sv = plsc.load_gather(rinv, [uiv[sl] + b * SU])
        tw16 = ebuf[sl] * sv
        twbuf[sl] = tw16
        eidx = jnp.arange(16, dtype=jnp.int32) + (g * 16)

        def scale_body(i, _):
          for k in range(2):
            dv = jnp.zeros((16,), jnp.int32) + (i * 2 + k)
            v = plsc.load_gather(rows, [eidx, dv]) * tw16
            plsc.store_scatter(rows, [eidx, dv], v)
          return 0
        lax.fori_loop(0, D // 2, scale_body, 0)
      pltpu.sync_copy(twbuf, tw_out.at[pl.ds(eoff, CH)])
      pltpu.sync_copy(rows, S_sh.at[iidx], add=True)
      return 0

    lax.fori_loop(0, NCHUNK, chunk_body, 0)
    plsc.subcore_barrier()

    def wbody(j, _):
      r0 = sid * rows_per_tile + j * 80
      pltpu.sync_copy(S_sh.at[pl.ds(r0, 80), :],
                      Sp_out.at[cid, pl.ds(b * SU + r0, 80), :])
      return 0
    lax.fori_loop(0, rows_per_tile // 80, wbody, 0)
    plsc.subcore_barrier()


def _final_gather(gu_ref, gi_ref, us_ref, po_ref, ne_ref,
                  ou, op_, on, idxv, rows):
  cid = lax.axis_index("c")
  sid = lax.axis_index("s")
  wid = cid * NS + sid
  base = pl.multiple_of(wid * (BATCH // NW), 8)
  for src_idx, tbl, dst in ((us_ref, gu_ref, ou), (po_ref, gi_ref, op_),
                            (ne_ref, gi_ref, on)):
    pltpu.sync_copy(src_idx.at[pl.ds(base, BATCH // NW)], idxv)
    pltpu.sync_copy(tbl.at[idxv], rows)
    pltpu.sync_copy(rows, dst.at[pl.ds(base, BATCH // NW), :])


def _dense_body(ut_ref, it_ref, sp_ref, wq_ref, bq_ref, wk_ref, bk_ref,
                wv_ref, bv_ref, wo_ref, bo_ref, g1_ref, gt_ref, lg_ref,
                lb_ref, gu_ref, gi_ref):
  wq = wq_ref[...]
  wk = wk_ref[...]
  wv = wv_ref[...]
  wo = wo_ref[...]
  g1 = g1_ref[...]
  gt = gt_ref[...]
  bq = bq_ref[...]
  bk = bk_ref[...]
  bv = bv_ref[...]
  bo = bo_ref[...]
  lg = lg_ref[...]
  lb = lb_ref[...]
  inv = np.float32(1.0 / np.sqrt(DH))

  def half(xs):
    qs = [jnp.dot(x, wq, preferred_element_type=jnp.float32) + bq for x in xs]
    ks = [jnp.dot(x, wk, preferred_element_type=jnp.float32) + bk for x in xs]
    vs = [jnp.dot(x, wv, preferred_element_type=jnp.float32) + bv for x in xs]
    ys = []
    for i in range(NB):
      sc = [jnp.dot(qs[i] * ks[j], g1, preferred_element_type=jnp.float32)
            * inv for j in range(NB)]
      m = jnp.maximum(jnp.maximum(sc[0], sc[1]), sc[2])
      ex = [jnp.exp(s - m) for s in sc]
      zden = ex[0] + ex[1] + ex[2]
      att = jnp.zeros_like(xs[i])
      for j in range(NB):
        att = att + jnp.dot(ex[j] / zden, gt,
                            preferred_element_type=jnp.float32) * vs[j]
      o = jnp.dot(att, wo, preferred_element_type=jnp.float32) + bo + xs[i]
      mu = jnp.mean(o, axis=-1, keepdims=True)
      cen = o - mu
      var = jnp.mean(cen * cen, axis=-1, keepdims=True)
      ys.append(cen * lax.rsqrt(var + 1e-5) * lg + lb)
    return ys[0] + ys[1] + ys[2]

  xu = ut_ref[...]
  gu_ref[...] = half([xu[b] for b in range(NB)])
  xi = it_ref[...]
  sp = sp_ref[...]
  gi_ref[...] = half([xi[b] + float(NL) * (sp[0, b] + sp[1, b])
                      for b in range(NB)])


def kernel(user_table, item_table, alpha_W, alpha_b, Wq, bq, Wk, bk, Wv, bv,
           Wo, bo, ln_g, ln_b, edge_index, users, pos_items, neg_items):
  f32 = jnp.float32
  ut3 = jnp.transpose(user_table, (1, 0, 2)).astype(f32)
  it3 = jnp.transpose(item_table, (1, 0, 2)).astype(f32)
  ut_flat = ut3.reshape(NB * NU, D)
  it_flat = it3.reshape(NB * NI, D)
  ei = edge_index.astype(jnp.int32)
  ui = ei[:, 0, :].reshape(NB * E)
  ii = ei[:, 1, :].reshape(NB * E)
  aWb = jnp.broadcast_to(alpha_W.astype(f32)[:, :, None],
                         (NB, D, 16)).reshape(NB * D, 16)
  ab16 = jnp.broadcast_to(alpha_b.astype(f32)[:, None], (NB, 16))

  p1 = pl.kernel(
      _edge_pass1,
      out_type=(
          jax.ShapeDtypeStruct((NB * E,), f32),
          jax.ShapeDtypeStruct((NB * E,), f32),
          jax.ShapeDtypeStruct((NC * NB * SU,), f32),
      ),
      mesh=_mesh,
      compiler_params=pltpu.CompilerParams(needs_layout_passes=False),
      scratch_types=[
          pltpu.VMEM((CH,), jnp.int32),
          pltpu.VMEM((CH,), jnp.int32),
          pltpu.VMEM((CH,), jnp.int32),
          pltpu.VMEM((CH,), jnp.int32),
          pltpu.VMEM((CH,), jnp.int32),
          pltpu.VMEM((CH, D), f32),
          pltpu.VMEM((CH, D), f32),
          pltpu.VMEM((CH,), f32),
          pltpu.VMEM((CH,), f32),
          pltpu.VMEM((NB * D, 16), f32),
          pltpu.VMEM((NB, 16), f32),
          pltpu.VMEM(((NB * SU) // NS,), f32),
          pltpu.VMEM_SHARED((NB * SU,), f32),
      ],
      name="swgcn_edge_pass1",
  )
  e_arr, aw_arr, sp_arr = p1(ut_flat, it_flat, ui, ii, aWb, ab16)

  p2 = pl.kernel(
      _edge_pass2,
      out_type=(
          jax.ShapeDtypeStruct((NB * E,), f32),
          jax.ShapeDtypeStruct((NC, NB * SU, D), f32),
      ),
      mesh=_mesh,
      compiler_params=pltpu.CompilerParams(needs_layout_passes=False),
      scratch_types=[
          pltpu.VMEM((CH,), jnp.int32),
          pltpu.VMEM((CH,), jnp.int32),
          pltpu.VMEM((CH,), jnp.int32),
          pltpu.VMEM((CH,), jnp.int32),
          pltpu.VMEM((CH, D), f32),
          pltpu.VMEM((CH,), f32),
          pltpu.VMEM((CH,), f32),
          pltpu.VMEM((NB * SU,), f32),
          pltpu.VMEM((NB * SU,), f32),
          pltpu.VMEM((80, D), f32),
          pltpu.VMEM_SHARED((SU, D), f32),
      ],
      name="swgcn_edge_pass2",
  )
  tw_arr, Sp_arr = p2(ut_flat, ui, ii, e_arr, sp_arr)

  Sp4 = Sp_arr.reshape(NC, NB, SU, D)

  grid = 5
  blk = NU // grid
  g1 = np.zeros((D, D), np.float32)
  gtm = np.zeros((D, D), np.float32)
  for h in range(H):
    g1[h * DH:(h + 1) * DH, h] = 1.0
    gtm[h, h * DH:(h + 1) * DH] = 1.0
  w2 = lambda w: w.astype(f32)
  b2 = lambda b: b.astype(f32).reshape(1, D)

  dense = pl.pallas_call(
      _dense_body,
      grid=(grid,),
      in_specs=[
          pl.BlockSpec((NB, blk, D), lambda i: (0, i, 0)),
          pl.BlockSpec((NB, blk, D), lambda i: (0, i, 0)),
          pl.BlockSpec((NC, NB, blk, D), lambda i: (0, 0, i, 0)),
          pl.BlockSpec((D, D), lambda i: (0, 0)),
          pl.BlockSpec((1, D), lambda i: (0, 0)),
          pl.BlockSpec((D, D), lambda i: (0, 0)),
          pl.BlockSpec((1, D), lambda i: (0, 0)),
          pl.BlockSpec((D, D), lambda i: (0, 0)),
          pl.BlockSpec((1, D), lambda i: (0, 0)),
          pl.BlockSpec((D, D), lambda i: (0, 0)),
          pl.BlockSpec((1, D), lambda i: (0, 0)),
          pl.BlockSpec((D, D), lambda i: (0, 0)),
          pl.BlockSpec((D, D), lambda i: (0, 0)),
          pl.BlockSpec((1, D), lambda i: (0, 0)),
          pl.BlockSpec((1, D), lambda i: (0, 0)),
      ],
      out_specs=[
          pl.BlockSpec((blk, D), lambda i: (i, 0)),
          pl.BlockSpec((blk, D), lambda i: (i, 0)),
      ],
      out_shape=[
          jax.ShapeDtypeStruct((NU, D), f32),
          jax.ShapeDtypeStruct((NI, D), f32),
      ],
      name="swgcn_dense",
  )
  g_u, g_i = dense(ut3, it3, Sp4, w2(Wq), b2(bq), w2(Wk), b2(bk), w2(Wv),
                   b2(bv), w2(Wo), b2(bo), jnp.asarray(g1), jnp.asarray(gtm),
                   b2(ln_g), b2(ln_b))

  p3 = pl.kernel(
      _final_gather,
      out_type=(
          jax.ShapeDtypeStruct((BATCH, D), f32),
          jax.ShapeDtypeStruct((BATCH, D), f32),
          jax.ShapeDtypeStruct((BATCH, D), f32),
      ),
      mesh=_mesh,
      compiler_params=pltpu.CompilerParams(needs_layout_passes=False),
      scratch_types=[
          pltpu.VMEM((BATCH // NW,), jnp.int32),
          pltpu.VMEM((BATCH // NW, D), f32),
      ],
      name="swgcn_final_gather",
  )
  o_u, o_p, o_n = p3(g_u, g_i, users.astype(jnp.int32),
                     pos_items.astype(jnp.int32), neg_items.astype(jnp.int32))
  return (o_u, o_p, o_n, tw_arr.reshape(NB, E), aw_arr.reshape(NB, E))
